# baseline (device time: 90910 ns/iter reference)
import jax
import jax.numpy as jnp
from jax import lax
from jax.experimental import pallas as pl
from jax.experimental.pallas import tpu as pltpu

B, S, H, Dh, Dr = 2, 512, 16, 128, 32
D = 2048
DC = 128
N_X = 2
HH = H // 2
HW = HH * Dh
HWR = HH * Dr
BS = B * S
SCALE = (Dh + Dr) ** -0.5
BF16 = jnp.bfloat16
F32 = jnp.float32

_VMEM = pl.BlockSpec(memory_space=pltpu.VMEM)
_MESH = pl.DeviceIdType.MESH


def _proj_body(x_ref, wq_ref, wqr_ref, wkr_ref, wdkv_ref,
               wuk_ref, wuv_ref,
               q_ref, qr_ref, kr_ref, kg_ref, vg_ref,
               cg, wukg, wuvg, send_sems, recv_sems):
    my_x = lax.axis_index("x")
    my_y = lax.axis_index("y")
    xpeer = (1 - my_x, my_y)

    barrier = pltpu.get_barrier_semaphore()
    pl.semaphore_signal(barrier, inc=1, device_id=xpeer, device_id_type=_MESH)
    pl.semaphore_wait(barrier, 1)

    xb = x_ref[...].astype(BF16)
    cg[my_x] = jnp.dot(xb, wdkv_ref[...].astype(BF16),
                       preferred_element_type=F32).astype(BF16)
    wukg[my_x] = wuk_ref[...].astype(BF16)
    wuvg[my_x] = wuv_ref[...].astype(BF16)

    rdmas = []
    for i, ref in enumerate((cg, wukg, wuvg)):
        rdma = pltpu.make_async_remote_copy(
            src_ref=ref.at[my_x], dst_ref=ref.at[my_x],
            send_sem=send_sems.at[i], recv_sem=recv_sems.at[i],
            device_id=xpeer, device_id_type=_MESH,
        )
        rdma.start()
        rdmas.append(rdma)

    q_ref[...] = jnp.dot(xb, wq_ref[...].astype(BF16),
                         preferred_element_type=F32).astype(BF16)
    qr_ref[...] = jnp.dot(xb, wqr_ref[...].astype(BF16),
                          preferred_element_type=F32).astype(BF16)
    kr_ref[...] = jnp.dot(xb, wkr_ref[...].astype(BF16),
                          preferred_element_type=F32).astype(BF16)

    for rdma in rdmas:
        rdma.wait()

    kg_ref[...] = (jnp.dot(cg[0], wukg[0], preferred_element_type=F32)
                   + jnp.dot(cg[1], wukg[1],
                             preferred_element_type=F32)).astype(BF16)
    vg_ref[...] = (jnp.dot(cg[0], wuvg[0], preferred_element_type=F32)
                   + jnp.dot(cg[1], wuvg[1],
                             preferred_element_type=F32)).astype(BF16)


def _attn_out_body(q_ref, k_ref, v_ref, qr_ref, kr_ref, wo_ref,
                   out_ref, og, send_sems, recv_sems):
    my_x = lax.axis_index("x")
    my_y = lax.axis_index("y")
    ypeer = (my_x, 1 - my_y)
    own = my_y
    rem = 1 - my_y

    barrier = pltpu.get_barrier_semaphore()
    pl.semaphore_signal(barrier, inc=1, device_id=ypeer, device_id_type=_MESH)
    pl.semaphore_wait(barrier, 1)

    rdmas = []
    for h in range(HH):
        cols = slice(h * Dh, (h + 1) * Dh)
        for b in range(B):
            rows = slice(b * S, (b + 1) * S)
            s = lax.dot_general(q_ref[rows, cols], k_ref[rows, cols],
                                (((1,), (1,)), ((), ())),
                                preferred_element_type=F32)
            s += lax.dot_general(qr_ref[rows, h * Dr:(h + 1) * Dr],
                                 kr_ref[rows, :],
                                 (((1,), (1,)), ((), ())),
                                 preferred_element_type=F32)
            s *= SCALE
            m = jnp.max(s, axis=1, keepdims=True)
            e = jnp.exp(s - m)
            p = (e / jnp.sum(e, axis=1, keepdims=True)).astype(BF16)
            og[own, rows, cols] = jnp.dot(p, v_ref[rows, cols],
                                          preferred_element_type=F32
                                          ).astype(BF16)
        rdma = pltpu.make_async_remote_copy(
            src_ref=og.at[own, :, cols],
            dst_ref=og.at[own, :, cols],
            send_sem=send_sems.at[h],
            recv_sem=recv_sems.at[h],
            device_id=ypeer, device_id_type=_MESH,
        )
        rdma.start()
        rdmas.append(rdma)

    out_ref[...] = jnp.dot(
        og[own], wo_ref[pl.ds(own * HW, HW), :].astype(BF16),
        preferred_element_type=F32)
    half = HH // 2
    for chunk in range(2):
        for h in range(chunk * half, (chunk + 1) * half):
            rdmas[h].wait_recv()
        ccols = pl.ds(chunk * half * Dh, half * Dh)
        out_ref[...] += jnp.dot(
            og[rem, :, ccols],
            wo_ref[pl.ds(rem * HW + chunk * half * Dh, half * Dh), :].astype(
                BF16),
            preferred_element_type=F32)
    for rdma in rdmas:
        rdma.wait_send()


def kernel(x, Wdkv, Wuk, Wuv, Wq, Wqr, Wkr, Wo):
    xb = x.reshape(BS, D)
    gy_ = lax.axis_index("y")

    wq_h = lax.dynamic_slice(Wq, (0, gy_ * HW), (D, HW))
    wqr_h = lax.dynamic_slice(Wqr, (0, gy_ * HWR), (D, HWR))
    wuk_h = lax.dynamic_slice(Wuk, (0, gy_ * HW), (DC, HW))
    wuv_h = lax.dynamic_slice(Wuv, (0, gy_ * HW), (DC, HW))

    q, qr, kr, kg, vg = pl.pallas_call(
        _proj_body,
        out_shape=(
            jax.ShapeDtypeStruct((BS, HW), BF16),
            jax.ShapeDtypeStruct((BS, HWR), BF16),
            jax.ShapeDtypeStruct((BS, Dr), BF16),
            jax.ShapeDtypeStruct((BS, HW), BF16),
            jax.ShapeDtypeStruct((BS, HW), BF16),
        ),
        in_specs=[_VMEM] * 7,
        out_specs=(_VMEM,) * 5,
        scratch_shapes=[
            pltpu.VMEM((N_X, BS, DC), BF16),
            pltpu.VMEM((N_X, DC, HW), BF16),
            pltpu.VMEM((N_X, DC, HW), BF16),
            pltpu.SemaphoreType.DMA((3,)),
            pltpu.SemaphoreType.DMA((3,)),
        ],
        compiler_params=pltpu.CompilerParams(
            collective_id=0, vmem_limit_bytes=100 * 1024 * 1024),
    )(xb, wq_h, wqr_h, Wkr, Wdkv, wuk_h, wuv_h)

    out = pl.pallas_call(
        _attn_out_body,
        out_shape=jax.ShapeDtypeStruct((BS, D), F32),
        in_specs=[_VMEM] * 6,
        out_specs=_VMEM,
        scratch_shapes=[
            pltpu.VMEM((2, BS, HW), BF16),
            pltpu.SemaphoreType.DMA((HH,)),
            pltpu.SemaphoreType.DMA((HH,)),
        ],
        compiler_params=pltpu.CompilerParams(
            collective_id=1, vmem_limit_bytes=100 * 1024 * 1024),
    )(q, kg, vg, qr, kr, Wo)
    return out.reshape(B, S, D)


# device time: 76474 ns/iter; 1.1888x vs baseline; 1.1888x over previous
import jax
import jax.numpy as jnp
from jax import lax
from jax.experimental import pallas as pl
from jax.experimental.pallas import tpu as pltpu

B, S, H, Dh, Dr = 2, 512, 16, 128, 32
D = 2048
DC = 128
N_X = 2
N_DEV = 4
HG = H // N_DEV
GW = HG * Dh
GWR = HG * Dr
BS = B * S
SCALE = (Dh + Dr) ** -0.5
BF16 = jnp.bfloat16
F32 = jnp.float32

_VMEM = pl.BlockSpec(memory_space=pltpu.VMEM)
_MESH = pl.DeviceIdType.MESH


def _proj_body(x_ref, wq_ref, wqr_ref, wkr_ref, wdkv_ref,
               wuk_ref, wuv_ref,
               q_ref, qr_ref, kr_ref, kg_ref, vg_ref,
               cg, wukg, wuvg, wuk_sb, wuv_sb, send_sems, recv_sems):
    my_x = lax.axis_index("x")
    my_y = lax.axis_index("y")
    g = 2 * my_x + my_y
    peer_g = 2 * (1 - my_x) + my_y
    xpeer = (1 - my_x, my_y)

    barrier = pltpu.get_barrier_semaphore()
    pl.semaphore_signal(barrier, inc=1, device_id=xpeer, device_id_type=_MESH)
    pl.semaphore_wait(barrier, 1)

    wdkv = wdkv_ref[...].astype(BF16)
    for b in range(B):
        cg[my_x, b * S:(b + 1) * S, :] = jnp.dot(
            x_ref[b].astype(BF16), wdkv,
            preferred_element_type=F32).astype(BF16)
    wukg[my_x] = wuk_ref[:, pl.ds(g * GW, GW)].astype(BF16)
    wuvg[my_x] = wuv_ref[:, pl.ds(g * GW, GW)].astype(BF16)
    wuk_sb[...] = wuk_ref[:, pl.ds(peer_g * GW, GW)].astype(BF16)
    wuv_sb[...] = wuv_ref[:, pl.ds(peer_g * GW, GW)].astype(BF16)

    rdmas = []
    for i, (src, dst) in enumerate((
            (cg.at[my_x], cg.at[my_x]),
            (wuk_sb, wukg.at[my_x]),
            (wuv_sb, wuvg.at[my_x]),
    )):
        rdma = pltpu.make_async_remote_copy(
            src_ref=src, dst_ref=dst,
            send_sem=send_sems.at[i], recv_sem=recv_sems.at[i],
            device_id=xpeer, device_id_type=_MESH,
        )
        rdma.start()
        rdmas.append(rdma)

    wq = wq_ref[:, pl.ds(g * GW, GW)].astype(BF16)
    wqr = wqr_ref[:, pl.ds(g * GWR, GWR)].astype(BF16)
    wkr = wkr_ref[...].astype(BF16)
    for b in range(B):
        rows = slice(b * S, (b + 1) * S)
        xb = x_ref[b].astype(BF16)
        q_ref[rows, :] = jnp.dot(xb, wq,
                                 preferred_element_type=F32).astype(BF16)
        qr_ref[rows, :] = jnp.dot(xb, wqr,
                                  preferred_element_type=F32).astype(BF16)
        kr_ref[rows, :] = jnp.dot(xb, wkr,
                                  preferred_element_type=F32).astype(BF16)

    for rdma in rdmas:
        rdma.wait()

    kg_ref[...] = (jnp.dot(cg[0], wukg[0], preferred_element_type=F32)
                   + jnp.dot(cg[1], wukg[1],
                             preferred_element_type=F32)).astype(BF16)
    vg_ref[...] = (jnp.dot(cg[0], wuvg[0], preferred_element_type=F32)
                   + jnp.dot(cg[1], wuvg[1],
                             preferred_element_type=F32)).astype(BF16)


def _attn_out_body(q_ref, k_ref, v_ref, qr_ref, kr_ref, wo_ref,
                   out_ref, og, send_sems, recv_sems):
    my_x = lax.axis_index("x")
    my_y = lax.axis_index("y")
    g = 2 * my_x + my_y
    gx = 2 * (1 - my_x) + my_y
    gy = 2 * my_x + (1 - my_y)
    gd = 2 * (1 - my_x) + (1 - my_y)
    peers = ((1 - my_x, my_y), (my_x, 1 - my_y), (1 - my_x, 1 - my_y))

    barrier = pltpu.get_barrier_semaphore()
    for p_id in peers:
        pl.semaphore_signal(barrier, inc=1, device_id=p_id,
                            device_id_type=_MESH)
    pl.semaphore_wait(barrier, 3)

    rdmas = []
    for h in range(HG):
        for b in range(B):
            rows = pl.ds(b * S, S)
            q = q_ref[rows, h * Dh:(h + 1) * Dh]
            k = k_ref[rows, h * Dh:(h + 1) * Dh]
            qr = qr_ref[rows, h * Dr:(h + 1) * Dr]
            s = lax.dot_general(q, k, (((1,), (1,)), ((), ())),
                                preferred_element_type=F32)
            s += lax.dot_general(qr, kr_ref[rows, :],
                                 (((1,), (1,)), ((), ())),
                                 preferred_element_type=F32)
            s *= SCALE
            m = jnp.max(s, axis=1, keepdims=True)
            e = jnp.exp(s - m)
            p = (e / jnp.sum(e, axis=1, keepdims=True)).astype(BF16)
            og[g, rows, h * Dh:(h + 1) * Dh] = jnp.dot(
                p, v_ref[rows, h * Dh:(h + 1) * Dh],
                preferred_element_type=F32).astype(BF16)
        for i, p_id in enumerate(peers):
            rdma = pltpu.make_async_remote_copy(
                src_ref=og.at[g, :, pl.ds(h * Dh, Dh)],
                dst_ref=og.at[g, :, pl.ds(h * Dh, Dh)],
                send_sem=send_sems.at[h * 3 + i],
                recv_sem=recv_sems.at[h * 3 + i],
                device_id=p_id, device_id_type=_MESH,
            )
            rdma.start()
            rdmas.append(rdma)

    def wo_slice(q_):
        return wo_ref[pl.ds(q_ * GW, GW), :].astype(BF16)

    w = wo_slice(g)
    for b in range(B):
        out_ref[b] = jnp.dot(og[g, b * S:(b + 1) * S, :], w,
                             preferred_element_type=F32)
    for i, slot in enumerate((gx, gy, gd)):
        for h in range(HG):
            rdmas[h * 3 + i].wait_recv()
        w = wo_slice(slot)
        for b in range(B):
            out_ref[b] += jnp.dot(og[slot, b * S:(b + 1) * S, :], w,
                                  preferred_element_type=F32)
    for rdma in rdmas:
        rdma.wait_send()


def kernel(x, Wdkv, Wuk, Wuv, Wq, Wqr, Wkr, Wo):
    q, qr, kr, kg, vg = pl.pallas_call(
        _proj_body,
        out_shape=(
            jax.ShapeDtypeStruct((BS, GW), BF16),
            jax.ShapeDtypeStruct((BS, GWR), BF16),
            jax.ShapeDtypeStruct((BS, Dr), BF16),
            jax.ShapeDtypeStruct((BS, GW), BF16),
            jax.ShapeDtypeStruct((BS, GW), BF16),
        ),
        in_specs=[_VMEM] * 7,
        out_specs=(_VMEM,) * 5,
        scratch_shapes=[
            pltpu.VMEM((N_X, BS, DC), BF16),
            pltpu.VMEM((N_X, DC, GW), BF16),
            pltpu.VMEM((N_X, DC, GW), BF16),
            pltpu.VMEM((DC, GW), BF16),
            pltpu.VMEM((DC, GW), BF16),
            pltpu.SemaphoreType.DMA((3,)),
            pltpu.SemaphoreType.DMA((3,)),
        ],
        compiler_params=pltpu.CompilerParams(
            collective_id=0, vmem_limit_bytes=100 * 1024 * 1024),
    )(x, Wq, Wqr, Wkr, Wdkv, Wuk, Wuv)

    out = pl.pallas_call(
        _attn_out_body,
        out_shape=jax.ShapeDtypeStruct((B, S, D), F32),
        in_specs=[_VMEM] * 6,
        out_specs=_VMEM,
        scratch_shapes=[
            pltpu.VMEM((N_DEV, BS, GW), BF16),
            pltpu.SemaphoreType.DMA((HG * 3,)),
            pltpu.SemaphoreType.DMA((HG * 3,)),
        ],
        compiler_params=pltpu.CompilerParams(
            collective_id=1, vmem_limit_bytes=100 * 1024 * 1024),
    )(q, kg, vg, qr, kr, Wo)
    return out
